# baseline (speedup 1.0000x reference)
import functools

import numpy as np
import jax
import jax.numpy as jnp
from jax import lax
from jax.experimental import pallas as pl
from jax.experimental.pallas import tpu as pltpu
from jax.experimental.pallas import tpu_sc as plsc

_MAX_SEQ_LEN = 2048
_D_MODEL = 768


def _sinusoidal_pe_np(max_len: int, d_model: int) -> np.ndarray:
    pos = np.arange(max_len, dtype=np.float32)[:, None]
    div = np.exp(
        np.arange(0, d_model, 2, dtype=np.float32) * (-np.log(10000.0) / d_model)
    )
    pe = np.zeros((max_len, d_model), dtype=np.float32)
    pe[:, 0::2] = np.sin(pos * div)
    pe[:, 1::2] = np.cos(pos * div)
    return pe


_PE = _sinusoidal_pe_np(_MAX_SEQ_LEN, _D_MODEL)

_NUM_CORES = 2
_NUM_SUBCORES = 16
_NW = _NUM_CORES * _NUM_SUBCORES
_LANES = 16


def _make_sc_kernel(N: int, S: int, D: int):
    b_per_w = N // _NW
    CH = 64
    n_ch = b_per_w // CH
    mesh = plsc.VectorSubcoreMesh(
        core_axis_name="c",
        subcore_axis_name="s",
        num_cores=_NUM_CORES,
        num_subcores=_NUM_SUBCORES,
    )

    @functools.partial(
        pl.kernel,
        out_type=jax.ShapeDtypeStruct((N, D), jnp.float32),
        mesh=mesh,
        scratch_types=[
            pltpu.VMEM((b_per_w,), jnp.int32),
            pltpu.VMEM((CH, D), jnp.float32),
            pltpu.VMEM((CH, D), jnp.float32),
            pltpu.SemaphoreType.DMA,
        ],
    )
    def run(xf_hbm, table_hbm, pe_hbm, out_hbm, idx_v, rows_v, pe_v, sem):
        wid = lax.axis_index("s") * _NUM_CORES + lax.axis_index("c")
        base = wid * b_per_w
        s_base = lax.rem(base, S)
        pltpu.sync_copy(xf_hbm.at[pl.ds(base, b_per_w)], idx_v)
        for c in range(n_ch):
            pltpu.sync_copy(pe_hbm.at[pl.ds(s_base + c * CH, CH)], pe_v)
            pltpu.async_copy(
                table_hbm.at[idx_v.at[pl.ds(c * CH, CH)]], rows_v, sem
            ).wait()

            def add_row(r, carry):
                for j in range(D // _LANES):
                    sl = pl.ds(j * _LANES, _LANES)
                    rows_v[r, sl] = rows_v[r, sl] + pe_v[r, sl]
                return carry

            lax.fori_loop(0, CH, add_row, 0)
            pltpu.sync_copy(rows_v, out_hbm.at[pl.ds(base + c * CH, CH)])

    return run


def kernel(x, tok_table):
    B, S = x.shape
    V, D = tok_table.shape
    N = B * S
    pe = jnp.asarray(_PE[:S])
    xf = x.reshape(N).astype(jnp.int32)
    run = _make_sc_kernel(N, S, D)
    out = run(xf, tok_table, pe)
    return out.reshape(B, S, D)


# speedup vs baseline: 1.1200x; 1.1200x over previous
import functools

import numpy as np
import jax
import jax.numpy as jnp
from jax import lax
from jax.experimental import pallas as pl
from jax.experimental.pallas import tpu as pltpu
from jax.experimental.pallas import tpu_sc as plsc

_MAX_SEQ_LEN = 2048
_D_MODEL = 768


def _sinusoidal_pe_np(max_len: int, d_model: int) -> np.ndarray:
    pos = np.arange(max_len, dtype=np.float32)[:, None]
    div = np.exp(
        np.arange(0, d_model, 2, dtype=np.float32) * (-np.log(10000.0) / d_model)
    )
    pe = np.zeros((max_len, d_model), dtype=np.float32)
    pe[:, 0::2] = np.sin(pos * div)
    pe[:, 1::2] = np.cos(pos * div)
    return pe


_PE = _sinusoidal_pe_np(_MAX_SEQ_LEN, _D_MODEL)

_NUM_CORES = 2
_NUM_SUBCORES = 16
_NW = _NUM_CORES * _NUM_SUBCORES
_LANES = 16


def _make_sc_kernel(N: int, S: int, D: int):
    b_per_w = N // _NW
    CH = 16
    n_ch = b_per_w // CH
    mesh = plsc.VectorSubcoreMesh(
        core_axis_name="c",
        subcore_axis_name="s",
        num_cores=_NUM_CORES,
        num_subcores=_NUM_SUBCORES,
    )

    @functools.partial(
        pl.kernel,
        out_type=jax.ShapeDtypeStruct((N, D), jnp.float32),
        mesh=mesh,
        scratch_types=[
            pltpu.VMEM((b_per_w,), jnp.int32),
            [pltpu.VMEM((CH, D), jnp.float32) for _ in range(2)],
            [pltpu.VMEM((CH, D), jnp.float32) for _ in range(2)],
            [pltpu.VMEM((CH, D), jnp.float32) for _ in range(2)],
            [pltpu.SemaphoreType.DMA for _ in range(2)],
            [pltpu.SemaphoreType.DMA for _ in range(2)],
            [pltpu.SemaphoreType.DMA for _ in range(2)],
        ],
    )
    def run(xf_hbm, table_hbm, pe_hbm, out_hbm,
            idx_v, rows_v, pe_v, out_v, gsem, psem, osem):
        wid = lax.axis_index("s") * _NUM_CORES + lax.axis_index("c")
        base = wid * b_per_w
        s_base = lax.rem(base, S)
        pltpu.sync_copy(xf_hbm.at[pl.ds(base, b_per_w)], idx_v)

        def issue_in(c):
            b = c % 2
            g = pltpu.async_copy(
                table_hbm.at[idx_v.at[pl.ds(c * CH, CH)]], rows_v[b], gsem[b]
            )
            p = pltpu.async_copy(
                pe_hbm.at[pl.ds(s_base + c * CH, CH)], pe_v[b], psem[b]
            )
            return g, p

        pending = {}
        out_pending = {}
        for c in range(min(2, n_ch)):
            pending[c] = issue_in(c)

        for c in range(n_ch):
            b = c % 2
            g, p = pending.pop(c)
            g.wait()
            p.wait()
            if c >= 2:
                out_pending.pop(c - 2).wait()

            def add_row(r, carry):
                for j in range(D // _LANES):
                    sl = pl.ds(j * _LANES, _LANES)
                    out_v[b][r, sl] = rows_v[b][r, sl] + pe_v[b][r, sl]
                return carry

            lax.fori_loop(0, CH, add_row, 0)
            if c + 2 < n_ch:
                pending[c + 2] = issue_in(c + 2)
            out_pending[c] = pltpu.async_copy(
                out_v[b], out_hbm.at[pl.ds(base + c * CH, CH)], osem[b]
            )
        for c in sorted(out_pending):
            out_pending.pop(c).wait()

    return run


def kernel(x, tok_table):
    B, S = x.shape
    V, D = tok_table.shape
    N = B * S
    pe = jnp.asarray(_PE[:S])
    xf = x.reshape(N).astype(jnp.int32)
    run = _make_sc_kernel(N, S, D)
    out = run(xf, tok_table, pe)
    return out.reshape(B, S, D)


# speedup vs baseline: 1.2217x; 1.0908x over previous
import functools

import numpy as np
import jax
import jax.numpy as jnp
from jax import lax
from jax.experimental import pallas as pl
from jax.experimental.pallas import tpu as pltpu
from jax.experimental.pallas import tpu_sc as plsc

_MAX_SEQ_LEN = 2048
_D_MODEL = 768


def _sinusoidal_pe_np(max_len: int, d_model: int) -> np.ndarray:
    pos = np.arange(max_len, dtype=np.float32)[:, None]
    div = np.exp(
        np.arange(0, d_model, 2, dtype=np.float32) * (-np.log(10000.0) / d_model)
    )
    pe = np.zeros((max_len, d_model), dtype=np.float32)
    pe[:, 0::2] = np.sin(pos * div)
    pe[:, 1::2] = np.cos(pos * div)
    return pe


_PE = _sinusoidal_pe_np(_MAX_SEQ_LEN, _D_MODEL)

_NUM_CORES = 2
_NUM_SUBCORES = 16
_NW = _NUM_CORES * _NUM_SUBCORES
_LANES = 16


def _make_sc_kernel(B: int, S: int, D: int):
    N = B * S
    b_per_w = N // _NW
    CH = 16
    n_ch = b_per_w // CH
    NB = 3
    mesh = plsc.VectorSubcoreMesh(
        core_axis_name="c",
        subcore_axis_name="s",
        num_cores=_NUM_CORES,
        num_subcores=_NUM_SUBCORES,
    )

    @functools.partial(
        pl.kernel,
        out_type=jax.ShapeDtypeStruct((B, S, D), jnp.float32),
        mesh=mesh,
        scratch_types=[
            pltpu.VMEM((b_per_w,), jnp.int32),
            [pltpu.VMEM((CH, D), jnp.float32) for _ in range(NB)],
            [pltpu.VMEM((CH, D), jnp.float32) for _ in range(NB)],
            [pltpu.VMEM((CH, D), jnp.float32) for _ in range(NB)],
            [pltpu.SemaphoreType.DMA for _ in range(NB)],
            [pltpu.SemaphoreType.DMA for _ in range(NB)],
            [pltpu.SemaphoreType.DMA for _ in range(NB)],
        ],
    )
    def run(x_hbm, table_hbm, pe_hbm, out_hbm,
            idx_v, rows_v, pe_v, out_v, gsem, psem, osem):
        wid = lax.axis_index("s") * _NUM_CORES + lax.axis_index("c")
        base = wid * b_per_w
        b_idx = lax.div(base, S)
        s_base = lax.rem(base, S)
        pltpu.sync_copy(x_hbm.at[b_idx, pl.ds(s_base, b_per_w)], idx_v)

        def issue_in(c):
            b = c % NB
            g = pltpu.async_copy(
                table_hbm.at[idx_v.at[pl.ds(c * CH, CH)]], rows_v[b], gsem[b]
            )
            p = pltpu.async_copy(
                pe_hbm.at[pl.ds(s_base + c * CH, CH)], pe_v[b], psem[b]
            )
            return g, p

        pending = {}
        out_pending = {}
        for c in range(min(NB, n_ch)):
            pending[c] = issue_in(c)

        for c in range(n_ch):
            b = c % NB
            g, p = pending.pop(c)
            g.wait()
            p.wait()
            if c >= NB:
                out_pending.pop(c - NB).wait()

            def add_row(r, carry):
                for j in range(D // _LANES):
                    sl = pl.ds(j * _LANES, _LANES)
                    out_v[b][r, sl] = rows_v[b][r, sl] + pe_v[b][r, sl]
                return carry

            lax.fori_loop(0, CH, add_row, 0)
            if c + NB < n_ch:
                pending[c + NB] = issue_in(c + NB)
            out_pending[c] = pltpu.async_copy(
                out_v[b],
                out_hbm.at[b_idx, pl.ds(s_base + c * CH, CH)],
                osem[b],
            )
        for c in sorted(out_pending):
            out_pending.pop(c).wait()

    return run


def kernel(x, tok_table):
    B, S = x.shape
    V, D = tok_table.shape
    pe = jnp.asarray(_PE[:S])
    run = _make_sc_kernel(B, S, D)
    return run(x, tok_table, pe)
